# baseline (speedup 1.0000x reference)
import functools

import jax
import jax.numpy as jnp
from jax import lax
from jax.experimental import pallas as pl
from jax.experimental.pallas import tpu as pltpu
from jax.experimental.pallas import tpu_sc as plsc

N = 10000
E = 320000
D = 128

NC = 2
NS = 16
TILES = NC * NS
DH = D // NC
C = 64
K = 160
EPT = K * C
EPAD = TILES * EPT
NPAD = 10240
RPT = NPAD // NS
BR = 1024
GRID = NPAD // BR


def _sc_mesh():
  return plsc.VectorSubcoreMesh(core_axis_name="c", subcore_axis_name="s")


_SC_PARAMS = pltpu.CompilerParams(needs_layout_passes=False,
                                  use_tc_tiling_on_sc=False)


def _sc_agg_rows(hp, src_t, dst_t, zrows):
  @functools.partial(
      pl.kernel,
      out_type=jax.ShapeDtypeStruct((NC, NPAD, D), jnp.float32),
      mesh=_sc_mesh(),
      compiler_params=_SC_PARAMS,
      scratch_types=[
          pltpu.VMEM((K, C), jnp.int32),
          pltpu.VMEM((K, C), jnp.int32),
          pltpu.VMEM((C, D), jnp.float32),
          pltpu.VMEM((C, D), jnp.float32),
          pltpu.VMEM_SHARED((NPAD, D), jnp.float32),
          pltpu.SemaphoreType.DMA,
          pltpu.SemaphoreType.DMA,
          pltpu.SemaphoreType.DMA,
          pltpu.SemaphoreType.DMA,
      ],
  )
  def k(hp_hbm, src_hbm, dst_hbm, z_hbm, out_hbm,
        sidx, didx, buf0, buf1, acc, sem0, sem1, sems0, sems1):
    c = lax.axis_index("c")
    s = lax.axis_index("s")
    w = c * NS + s
    pltpu.sync_copy(src_hbm.at[w], sidx)
    pltpu.sync_copy(dst_hbm.at[w], didx)
    pltpu.sync_copy(z_hbm, acc.at[pl.ds(s * RPT, RPT)])
    plsc.subcore_barrier()

    def startg(j, buf, sem):
      pltpu.async_copy(hp_hbm.at[sidx.at[j]], buf, sem)

    def waitg(j, buf, sem):
      pltpu.make_async_copy(hp_hbm.at[sidx.at[j]], buf, sem).wait()

    def starts(j, buf, sem):
      pltpu.async_copy(buf, acc.at[didx.at[j]], sem, add=True)

    def waits(j, buf, sem):
      pltpu.make_async_copy(buf, acc.at[didx.at[j]], sem).wait()

    startg(0, buf0, sem0)
    startg(1, buf1, sem1)

    @pl.loop(0, K, step=2)
    def _(g):
      waitg(g, buf0, sem0)
      starts(g, buf0, sems0)
      waitg(g + 1, buf1, sem1)
      starts(g + 1, buf1, sems1)

      @pl.when(g + 2 < K)
      def _():
        waits(g, buf0, sems0)
        startg(g + 2, buf0, sem0)

      @pl.when(g + 3 < K)
      def _():
        waits(g + 1, buf1, sems1)
        startg(g + 3, buf1, sem1)

    waits(K - 2, buf0, sems0)
    waits(K - 1, buf1, sems1)
    plsc.subcore_barrier()
    pltpu.sync_copy(acc.at[pl.ds(s * RPT, RPT)],
                    out_hbm.at[c].at[pl.ds(s * RPT, RPT)])

  return k(hp, src_t, dst_t, zrows)


def _sc_degree(dst_flat, zcol):
  @functools.partial(
      pl.kernel,
      out_type=jax.ShapeDtypeStruct((TILES, NPAD), jnp.float32),
      mesh=_sc_mesh(),
      compiler_params=_SC_PARAMS,
      scratch_types=[
          pltpu.VMEM((EPT,), jnp.int32),
          pltpu.VMEM((NPAD,), jnp.float32),
      ],
  )
  def k(dst_hbm, z_hbm, out_hbm, didx, acc):
    c = lax.axis_index("c")
    s = lax.axis_index("s")
    w = c * NS + s
    pltpu.sync_copy(dst_hbm.at[w], didx)
    pltpu.sync_copy(z_hbm, acc)
    ones = jnp.ones((16,), jnp.float32)

    @pl.loop(0, EPT // 16)
    def _(i):
      d = didx[pl.ds(i * 16, 16)]
      plsc.addupdate_scatter(acc, [d], ones)

    pltpu.sync_copy(acc, out_hbm.at[w])

  return k(dst_flat, zcol)


def _sc_agg_scalar(vals, src_flat, dst_flat, zcol):
  @functools.partial(
      pl.kernel,
      out_type=jax.ShapeDtypeStruct((TILES, NPAD), jnp.float32),
      mesh=_sc_mesh(),
      compiler_params=_SC_PARAMS,
      scratch_types=[
          pltpu.VMEM((EPT,), jnp.int32),
          pltpu.VMEM((EPT,), jnp.int32),
          pltpu.VMEM((NPAD,), jnp.float32),
          pltpu.VMEM((NPAD,), jnp.float32),
      ],
  )
  def k(vals_hbm, src_hbm, dst_hbm, z_hbm, out_hbm, sidx, didx, vloc, acc):
    c = lax.axis_index("c")
    s = lax.axis_index("s")
    w = c * NS + s
    pltpu.sync_copy(src_hbm.at[w], sidx)
    pltpu.sync_copy(dst_hbm.at[w], didx)
    pltpu.sync_copy(vals_hbm, vloc)
    pltpu.sync_copy(z_hbm, acc)

    @pl.loop(0, EPT // 16)
    def _(i):
      ss = sidx[pl.ds(i * 16, 16)]
      dd = didx[pl.ds(i * 16, 16)]
      v = plsc.load_gather(vloc, [ss])
      plsc.addupdate_scatter(acc, [dd], v)

    pltpu.sync_copy(acc, out_hbm.at[w])

  return k(vals, src_flat, dst_flat, zcol)


def _tc_first(deg_parts, x_pad, w1):
  def body(deg_ref, x_ref, w_ref, h_ref, dinv_ref):
    deg = jnp.sum(deg_ref[...], axis=0) + 1.0
    dinv = 1.0 / jnp.sqrt(deg)
    h = jnp.dot(x_ref[...], w_ref[...], preferred_element_type=jnp.float32)
    h_ref[...] = h * dinv[:, None]
    dinv_ref[...] = dinv

  return pl.pallas_call(
      body,
      grid=(GRID,),
      in_specs=[
          pl.BlockSpec((TILES, BR), lambda i: (0, i)),
          pl.BlockSpec((BR, D), lambda i: (i, 0)),
          pl.BlockSpec((D, D), lambda i: (0, 0)),
      ],
      out_specs=[
          pl.BlockSpec((BR, D), lambda i: (i, 0)),
          pl.BlockSpec((BR,), lambda i: (i,)),
      ],
      out_shape=[
          jax.ShapeDtypeStruct((NPAD, D), jnp.float32),
          jax.ShapeDtypeStruct((NPAD,), jnp.float32),
      ],
  )(deg_parts, x_pad, w1)


def _tc_mid(parts, hp, dinv, b2d, w, dn):
  def body(parts_ref, hp_ref, dinv_ref, b_ref, w_ref, out_ref):
    dinv = dinv_ref[...]
    t = parts_ref[0] + parts_ref[1] + hp_ref[...]
    t = t * dinv[:, None] + b_ref[...]
    t = jnp.maximum(t, 0.0)
    r = jnp.dot(t, w_ref[...], preferred_element_type=jnp.float32)
    if dn == 1:
      out_ref[...] = r[:, 0] * dinv
    else:
      out_ref[...] = r * dinv[:, None]

  out_shape = (NPAD, dn) if dn > 1 else (NPAD,)
  out_spec = (pl.BlockSpec((BR, dn), lambda i: (i, 0)) if dn > 1
              else pl.BlockSpec((BR,), lambda i: (i,)))
  return pl.pallas_call(
      body,
      grid=(GRID,),
      in_specs=[
          pl.BlockSpec((NC, BR, D), lambda i: (0, i, 0)),
          pl.BlockSpec((BR, D), lambda i: (i, 0)),
          pl.BlockSpec((BR,), lambda i: (i,)),
          pl.BlockSpec((1, D), lambda i: (0, 0)),
          pl.BlockSpec((D, dn), lambda i: (0, 0)),
      ],
      out_specs=out_spec,
      out_shape=jax.ShapeDtypeStruct(out_shape, jnp.float32),
  )(parts, hp, dinv, b2d, w)


def _tc_final(parts, h3p, dinv, b3):
  def body(parts_ref, h3_ref, dinv_ref, b_ref, out_ref):
    agg = jnp.sum(parts_ref[...], axis=0)
    out_ref[...] = dinv_ref[...] * (agg + h3_ref[...]) + b_ref[0]

  return pl.pallas_call(
      body,
      out_shape=jax.ShapeDtypeStruct((NPAD,), jnp.float32),
  )(parts, h3p, dinv, b3)


def kernel(x, edge_index, W1, b1, W2, b2, W3, b3):
  f32 = jnp.float32
  src = edge_index[0].astype(jnp.int32)
  dst = edge_index[1].astype(jnp.int32)
  src_pad = jnp.concatenate([src, jnp.zeros((EPAD - E,), jnp.int32)])
  pad_dst = N + (jnp.arange(EPAD - E, dtype=jnp.int32) % 16)
  dst_pad = jnp.concatenate([dst, pad_dst])
  src_t = src_pad.reshape(TILES, K, C)
  dst_t = dst_pad.reshape(TILES, K, C)
  src_flat = src_pad.reshape(TILES, EPT)
  dst_flat = dst_pad.reshape(TILES, EPT)

  x_pad = jnp.zeros((NPAD, D), f32).at[:N].set(x)
  zrows = jnp.zeros((RPT, D), f32)
  zcol = jnp.zeros((NPAD,), f32)
  b1_2d = b1.reshape(1, D)
  b2_2d = b2.reshape(1, D)

  deg_parts = _sc_degree(dst_flat, zcol)
  h1p, dinv = _tc_first(deg_parts, x_pad, W1)
  parts1 = _sc_agg_rows(h1p, src_t, dst_t, zrows)
  h2p = _tc_mid(parts1, h1p, dinv, b1_2d, W2, D)
  parts2 = _sc_agg_rows(h2p, src_t, dst_t, zrows)
  h3p = _tc_mid(parts2, h2p, dinv, b2_2d, W3, 1)
  parts3 = _sc_agg_scalar(h3p, src_flat, dst_flat, zcol)
  out = _tc_final(parts3, h3p, dinv, b3)
  return out[:N]


# speedup vs baseline: 2.1335x; 2.1335x over previous
import functools

import jax
import jax.numpy as jnp
from jax import lax
from jax.experimental import pallas as pl
from jax.experimental.pallas import tpu as pltpu
from jax.experimental.pallas import tpu_sc as plsc

N = 10000
E = 320000
D = 128

NC = 2
NS = 16
TILES = NC * NS
DH = D // NC
C = 128
K = 80
K2 = 160
C4 = 64
K4 = 320
EPT = K * C
EPAD = TILES * EPT
NPAD = 10240
RPT = NPAD // NS
BR = 1024
GRID = NPAD // BR


def _sc_mesh():
  return plsc.VectorSubcoreMesh(core_axis_name="c", subcore_axis_name="s")


_SC_PARAMS = pltpu.CompilerParams(needs_layout_passes=False,
                                  use_tc_tiling_on_sc=False)


def _sc_agg_rows(hp2, src_t, dst_t, zrows):
  @functools.partial(
      pl.kernel,
      out_type=jax.ShapeDtypeStruct((NC, NPAD, DH), jnp.float32),
      mesh=_sc_mesh(),
      compiler_params=_SC_PARAMS,
      scratch_types=[
          pltpu.VMEM((K4, C4), jnp.int32),
          pltpu.VMEM((K4, C4), jnp.int32),
          pltpu.VMEM((C4, DH), jnp.float32),
          pltpu.VMEM((C4, DH), jnp.float32),
          pltpu.VMEM_SHARED((NPAD, DH), jnp.float32),
          pltpu.VMEM_SHARED((NPAD, DH), jnp.float32),
          pltpu.SemaphoreType.DMA,
          pltpu.SemaphoreType.DMA,
      ],
  )
  def k(hp_hbm, src_hbm, dst_hbm, z_hbm, out_hbm,
        sidx, didx, buf0, buf1, hp_sh, acc, sem0, sem1):
    c = lax.axis_index("c")
    s = lax.axis_index("s")
    pltpu.sync_copy(src_hbm.at[s], sidx)
    pltpu.sync_copy(dst_hbm.at[s], didx)
    pltpu.sync_copy(hp_hbm.at[c].at[pl.ds(s * RPT, RPT)],
                    hp_sh.at[pl.ds(s * RPT, RPT)])
    pltpu.sync_copy(z_hbm, acc.at[pl.ds(s * RPT, RPT)])
    plsc.subcore_barrier()

    def start(j, buf, sem):
      pltpu.async_copy(hp_sh.at[sidx.at[j]], buf, sem)

    def wait(j, buf, sem):
      pltpu.make_async_copy(hp_sh.at[sidx.at[j]], buf, sem).wait()

    start(0, buf0, sem0)
    start(1, buf1, sem1)

    @pl.loop(0, K4, step=2)
    def _(g):
      wait(g, buf0, sem0)
      pltpu.sync_copy(buf0, acc.at[didx.at[g]], add=True)

      @pl.when(g + 2 < K4)
      def _():
        start(g + 2, buf0, sem0)

      wait(g + 1, buf1, sem1)
      pltpu.sync_copy(buf1, acc.at[didx.at[g + 1]], add=True)

      @pl.when(g + 3 < K4)
      def _():
        start(g + 3, buf1, sem1)

    plsc.subcore_barrier()
    pltpu.sync_copy(acc.at[pl.ds(s * RPT, RPT)],
                    out_hbm.at[c].at[pl.ds(s * RPT, RPT)])

  return k(hp2, src_t, dst_t, zrows)


def _sc_degree(dst_flat, zcol):
  @functools.partial(
      pl.kernel,
      out_type=jax.ShapeDtypeStruct((TILES, NPAD), jnp.float32),
      mesh=_sc_mesh(),
      compiler_params=_SC_PARAMS,
      scratch_types=[
          pltpu.VMEM((EPT,), jnp.int32),
          pltpu.VMEM((NPAD,), jnp.float32),
      ],
  )
  def k(dst_hbm, z_hbm, out_hbm, didx, acc):
    c = lax.axis_index("c")
    s = lax.axis_index("s")
    w = c * NS + s
    pltpu.sync_copy(dst_hbm.at[w], didx)
    pltpu.sync_copy(z_hbm, acc)
    ones = jnp.ones((16,), jnp.float32)

    @pl.loop(0, EPT // 16)
    def _(i):
      d = didx[pl.ds(i * 16, 16)]
      plsc.addupdate_scatter(acc, [d], ones)

    pltpu.sync_copy(acc, out_hbm.at[w])

  return k(dst_flat, zcol)


def _sc_agg_scalar(vals, src_flat, dst_flat, zcol):
  @functools.partial(
      pl.kernel,
      out_type=jax.ShapeDtypeStruct((TILES, NPAD), jnp.float32),
      mesh=_sc_mesh(),
      compiler_params=_SC_PARAMS,
      scratch_types=[
          pltpu.VMEM((EPT,), jnp.int32),
          pltpu.VMEM((EPT,), jnp.int32),
          pltpu.VMEM((NPAD,), jnp.float32),
          pltpu.VMEM((NPAD,), jnp.float32),
      ],
  )
  def k(vals_hbm, src_hbm, dst_hbm, z_hbm, out_hbm, sidx, didx, vloc, acc):
    c = lax.axis_index("c")
    s = lax.axis_index("s")
    w = c * NS + s
    pltpu.sync_copy(src_hbm.at[w], sidx)
    pltpu.sync_copy(dst_hbm.at[w], didx)
    pltpu.sync_copy(vals_hbm, vloc)
    pltpu.sync_copy(z_hbm, acc)

    @pl.loop(0, EPT // 16)
    def _(i):
      ss = sidx[pl.ds(i * 16, 16)]
      dd = didx[pl.ds(i * 16, 16)]
      v = plsc.load_gather(vloc, [ss])
      plsc.addupdate_scatter(acc, [dd], v)

    pltpu.sync_copy(acc, out_hbm.at[w])

  return k(vals, src_flat, dst_flat, zcol)


def _tc_first(deg_parts, x_pad, w1):
  def body(deg_ref, x_ref, w_ref, h_ref, dinv_ref):
    deg = jnp.sum(deg_ref[...], axis=0) + 1.0
    dinv = 1.0 / jnp.sqrt(deg)
    h = jnp.dot(x_ref[...], w_ref[...], preferred_element_type=jnp.float32)
    h = h * dinv[:, None]
    h_ref[0] = h[:, :DH]
    h_ref[1] = h[:, DH:]
    dinv_ref[...] = dinv

  return pl.pallas_call(
      body,
      grid=(GRID,),
      in_specs=[
          pl.BlockSpec((TILES, BR), lambda i: (0, i)),
          pl.BlockSpec((BR, D), lambda i: (i, 0)),
          pl.BlockSpec((D, D), lambda i: (0, 0)),
      ],
      out_specs=[
          pl.BlockSpec((NC, BR, DH), lambda i: (0, i, 0)),
          pl.BlockSpec((BR,), lambda i: (i,)),
      ],
      out_shape=[
          jax.ShapeDtypeStruct((NC, NPAD, DH), jnp.float32),
          jax.ShapeDtypeStruct((NPAD,), jnp.float32),
      ],
  )(deg_parts, x_pad, w1)


def _tc_mid(agg2, hp2, dinv, b2d, w, dn):
  def body(agg_ref, hp_ref, dinv_ref, b_ref, w_ref, out_ref):
    dinv = dinv_ref[...]
    ta = agg_ref[0] + hp_ref[0]
    tb = agg_ref[1] + hp_ref[1]
    ta = ta * dinv[:, None] + b_ref[0, :DH][None, :]
    tb = tb * dinv[:, None] + b_ref[0, DH:][None, :]
    ta = jnp.maximum(ta, 0.0)
    tb = jnp.maximum(tb, 0.0)
    r = (jnp.dot(ta, w_ref[...][:DH], preferred_element_type=jnp.float32)
         + jnp.dot(tb, w_ref[...][DH:], preferred_element_type=jnp.float32))
    if dn == 1:
      out_ref[...] = r[:, 0] * dinv
    else:
      r = r * dinv[:, None]
      out_ref[0] = r[:, :DH]
      out_ref[1] = r[:, DH:]

  out_shape = (NC, NPAD, DH) if dn > 1 else (NPAD,)
  out_spec = (pl.BlockSpec((NC, BR, DH), lambda i: (0, i, 0)) if dn > 1
              else pl.BlockSpec((BR,), lambda i: (i,)))
  return pl.pallas_call(
      body,
      grid=(GRID,),
      in_specs=[
          pl.BlockSpec((NC, BR, DH), lambda i: (0, i, 0)),
          pl.BlockSpec((NC, BR, DH), lambda i: (0, i, 0)),
          pl.BlockSpec((BR,), lambda i: (i,)),
          pl.BlockSpec((1, D), lambda i: (0, 0)),
          pl.BlockSpec((D, dn), lambda i: (0, 0)),
      ],
      out_specs=out_spec,
      out_shape=jax.ShapeDtypeStruct(out_shape, jnp.float32),
  )(agg2, hp2, dinv, b2d, w)


def _tc_final(parts, h3p, dinv, b3):
  def body(parts_ref, h3_ref, dinv_ref, b_ref, out_ref):
    agg = jnp.sum(parts_ref[...], axis=0)
    out_ref[...] = dinv_ref[...] * (agg + h3_ref[...]) + b_ref[0]

  return pl.pallas_call(
      body,
      out_shape=jax.ShapeDtypeStruct((NPAD,), jnp.float32),
  )(parts, h3p, dinv, b3)


def kernel(x, edge_index, W1, b1, W2, b2, W3, b3):
  f32 = jnp.float32
  src = edge_index[0].astype(jnp.int32)
  dst = edge_index[1].astype(jnp.int32)
  src_pad = jnp.concatenate([src, jnp.zeros((EPAD - E,), jnp.int32)])
  dst_pad = jnp.concatenate([dst, jnp.full((EPAD - E,), N, jnp.int32)])
  src_t = src_pad.reshape(NS, K4, C4)
  dst_t = dst_pad.reshape(NS, K4, C4)
  src_flat = src_pad.reshape(TILES, EPT)
  dst_flat = dst_pad.reshape(TILES, EPT)

  x_pad = jnp.zeros((NPAD, D), f32).at[:N].set(x)
  zrows = jnp.zeros((RPT, DH), f32)
  zcol = jnp.zeros((NPAD,), f32)
  b1_2d = b1.reshape(1, D)
  b2_2d = b2.reshape(1, D)

  deg_parts = _sc_degree(dst_flat, zcol)
  h1p, dinv = _tc_first(deg_parts, x_pad, W1)
  parts1 = _sc_agg_rows(h1p, src_t, dst_t, zrows)
  h2p = _tc_mid(parts1, h1p, dinv, b1_2d, W2, D)
  parts2 = _sc_agg_rows(h2p, src_t, dst_t, zrows)
  h3p = _tc_mid(parts2, h2p, dinv, b2_2d, W3, 1)
  parts3 = _sc_agg_scalar(h3p, src_flat, dst_flat, zcol)
  out = _tc_final(parts3, h3p, dinv, b3)
  return out[:N]


# speedup vs baseline: 2.1362x; 1.0013x over previous
import functools

import jax
import jax.numpy as jnp
from jax import lax
from jax.experimental import pallas as pl
from jax.experimental.pallas import tpu as pltpu
from jax.experimental.pallas import tpu_sc as plsc

N = 10000
E = 320000
D = 128

NC = 2
NS = 16
TILES = NC * NS
DH = D // NC
C4 = 80
K4 = 250
EPT = E // TILES
RPT = N // NS
BR = 1000
GRID = N // BR


def _sc_mesh():
  return plsc.VectorSubcoreMesh(core_axis_name="c", subcore_axis_name="s")


_SC_PARAMS = pltpu.CompilerParams(needs_layout_passes=False,
                                  use_tc_tiling_on_sc=False)


def _sc_agg_rows(hp2, src_t, dst_t, zrows):
  @functools.partial(
      pl.kernel,
      out_type=jax.ShapeDtypeStruct((NC, N, DH), jnp.float32),
      mesh=_sc_mesh(),
      compiler_params=_SC_PARAMS,
      scratch_types=[
          pltpu.VMEM((K4, C4), jnp.int32),
          pltpu.VMEM((K4, C4), jnp.int32),
          pltpu.VMEM((C4, DH), jnp.float32),
          pltpu.VMEM((C4, DH), jnp.float32),
          pltpu.VMEM_SHARED((N, DH), jnp.float32),
          pltpu.VMEM_SHARED((N, DH), jnp.float32),
          pltpu.SemaphoreType.DMA,
          pltpu.SemaphoreType.DMA,
      ],
  )
  def k(hp_hbm, src_hbm, dst_hbm, z_hbm, out_hbm,
        sidx, didx, buf0, buf1, hp_sh, acc, sem0, sem1):
    c = lax.axis_index("c")
    s = lax.axis_index("s")
    pltpu.sync_copy(src_hbm.at[s], sidx)
    pltpu.sync_copy(dst_hbm.at[s], didx)
    pltpu.sync_copy(hp_hbm.at[c].at[pl.ds(s * RPT, RPT)],
                    hp_sh.at[pl.ds(s * RPT, RPT)])
    pltpu.sync_copy(z_hbm, acc.at[pl.ds(s * RPT, RPT)])
    plsc.subcore_barrier()

    def start(j, buf, sem):
      pltpu.async_copy(hp_sh.at[sidx.at[j]], buf, sem)

    def wait(j, buf, sem):
      pltpu.make_async_copy(hp_sh.at[sidx.at[j]], buf, sem).wait()

    start(0, buf0, sem0)
    start(1, buf1, sem1)

    @pl.loop(0, K4, step=2)
    def _(g):
      wait(g, buf0, sem0)
      pltpu.sync_copy(buf0, acc.at[didx.at[g]], add=True)

      @pl.when(g + 2 < K4)
      def _():
        start(g + 2, buf0, sem0)

      wait(g + 1, buf1, sem1)
      pltpu.sync_copy(buf1, acc.at[didx.at[g + 1]], add=True)

      @pl.when(g + 3 < K4)
      def _():
        start(g + 3, buf1, sem1)

    plsc.subcore_barrier()
    pltpu.sync_copy(acc.at[pl.ds(s * RPT, RPT)],
                    out_hbm.at[c].at[pl.ds(s * RPT, RPT)])

  return k(hp2, src_t, dst_t, zrows)


def _sc_degree(dst_flat, zcol):
  @functools.partial(
      pl.kernel,
      out_type=jax.ShapeDtypeStruct((TILES, N), jnp.float32),
      mesh=_sc_mesh(),
      compiler_params=_SC_PARAMS,
      scratch_types=[
          pltpu.VMEM((EPT,), jnp.int32),
          pltpu.VMEM((N,), jnp.float32),
      ],
  )
  def k(dst_hbm, z_hbm, out_hbm, didx, acc):
    c = lax.axis_index("c")
    s = lax.axis_index("s")
    w = c * NS + s
    pltpu.sync_copy(dst_hbm.at[w], didx)
    pltpu.sync_copy(z_hbm, acc)
    ones = jnp.ones((16,), jnp.float32)

    @pl.loop(0, EPT // 16)
    def _(i):
      d = didx[pl.ds(i * 16, 16)]
      plsc.addupdate_scatter(acc, [d], ones)

    pltpu.sync_copy(acc, out_hbm.at[w])

  return k(dst_flat, zcol)


def _sc_agg_scalar(vals, src_flat, dst_flat, zcol):
  @functools.partial(
      pl.kernel,
      out_type=jax.ShapeDtypeStruct((TILES, N), jnp.float32),
      mesh=_sc_mesh(),
      compiler_params=_SC_PARAMS,
      scratch_types=[
          pltpu.VMEM((EPT,), jnp.int32),
          pltpu.VMEM((EPT,), jnp.int32),
          pltpu.VMEM((N,), jnp.float32),
          pltpu.VMEM((N,), jnp.float32),
      ],
  )
  def k(vals_hbm, src_hbm, dst_hbm, z_hbm, out_hbm, sidx, didx, vloc, acc):
    c = lax.axis_index("c")
    s = lax.axis_index("s")
    w = c * NS + s
    pltpu.sync_copy(src_hbm.at[w], sidx)
    pltpu.sync_copy(dst_hbm.at[w], didx)
    pltpu.sync_copy(vals_hbm, vloc)
    pltpu.sync_copy(z_hbm, acc)

    @pl.loop(0, EPT // 16)
    def _(i):
      ss = sidx[pl.ds(i * 16, 16)]
      dd = didx[pl.ds(i * 16, 16)]
      v = plsc.load_gather(vloc, [ss])
      plsc.addupdate_scatter(acc, [dd], v)

    pltpu.sync_copy(acc, out_hbm.at[w])

  return k(vals, src_flat, dst_flat, zcol)


def _tc_first(deg_t, x, w1):
  def body(deg_ref, x_ref, w_ref, h_ref, dinv_ref):
    deg = jnp.sum(deg_ref[...], axis=1, keepdims=True) + 1.0
    dinv = 1.0 / jnp.sqrt(deg)
    h = jnp.dot(x_ref[...], w_ref[...], preferred_element_type=jnp.float32)
    h = h * dinv
    h_ref[0] = h[:, :DH]
    h_ref[1] = h[:, DH:]
    dinv_ref[...] = dinv

  return pl.pallas_call(
      body,
      grid=(GRID,),
      in_specs=[
          pl.BlockSpec((BR, TILES), lambda i: (i, 0)),
          pl.BlockSpec((BR, D), lambda i: (i, 0)),
          pl.BlockSpec((D, D), lambda i: (0, 0)),
      ],
      out_specs=[
          pl.BlockSpec((NC, BR, DH), lambda i: (0, i, 0)),
          pl.BlockSpec((BR, 1), lambda i: (i, 0)),
      ],
      out_shape=[
          jax.ShapeDtypeStruct((NC, N, DH), jnp.float32),
          jax.ShapeDtypeStruct((N, 1), jnp.float32),
      ],
  )(deg_t, x, w1)


def _tc_mid(agg2, hp2, dinv, b2d, w, dn):
  def body(agg_ref, hp_ref, dinv_ref, b_ref, w_ref, out_ref):
    dinv = dinv_ref[...]
    ta = agg_ref[0] + hp_ref[0]
    tb = agg_ref[1] + hp_ref[1]
    ta = ta * dinv + b_ref[0, :DH][None, :]
    tb = tb * dinv + b_ref[0, DH:][None, :]
    ta = jnp.maximum(ta, 0.0)
    tb = jnp.maximum(tb, 0.0)
    r = (jnp.dot(ta, w_ref[...][:DH], preferred_element_type=jnp.float32)
         + jnp.dot(tb, w_ref[...][DH:], preferred_element_type=jnp.float32))
    r = r * dinv
    if dn == 1:
      out_ref[...] = r
    else:
      out_ref[0] = r[:, :DH]
      out_ref[1] = r[:, DH:]

  out_shape = (NC, N, DH) if dn > 1 else (N, 1)
  out_spec = (pl.BlockSpec((NC, BR, DH), lambda i: (0, i, 0)) if dn > 1
              else pl.BlockSpec((BR, 1), lambda i: (i, 0)))
  return pl.pallas_call(
      body,
      grid=(GRID,),
      in_specs=[
          pl.BlockSpec((NC, BR, DH), lambda i: (0, i, 0)),
          pl.BlockSpec((NC, BR, DH), lambda i: (0, i, 0)),
          pl.BlockSpec((BR, 1), lambda i: (i, 0)),
          pl.BlockSpec((1, D), lambda i: (0, 0)),
          pl.BlockSpec((D, dn), lambda i: (0, 0)),
      ],
      out_specs=out_spec,
      out_shape=jax.ShapeDtypeStruct(out_shape, jnp.float32),
  )(agg2, hp2, dinv, b2d, w)


def _tc_final(parts, h3p, dinv, b3):
  def body(parts_ref, h3_ref, dinv_ref, b_ref, out_ref):
    agg = jnp.sum(parts_ref[...], axis=0)
    out_ref[...] = dinv_ref[...][:, 0] * (agg + h3_ref[...]) + b_ref[0]

  return pl.pallas_call(
      body,
      out_shape=jax.ShapeDtypeStruct((N,), jnp.float32),
  )(parts, h3p, dinv, b3)


def kernel(x, edge_index, W1, b1, W2, b2, W3, b3):
  f32 = jnp.float32
  src = edge_index[0].astype(jnp.int32)
  dst = edge_index[1].astype(jnp.int32)
  src_t = src.reshape(NS, K4, C4)
  dst_t = dst.reshape(NS, K4, C4)
  src_flat = src.reshape(TILES, EPT)
  dst_flat = dst.reshape(TILES, EPT)

  zrows = jnp.zeros((RPT, DH), f32)
  zcol = jnp.zeros((N,), f32)
  b1_2d = b1.reshape(1, D)
  b2_2d = b2.reshape(1, D)

  deg_t = _sc_degree(dst_flat, zcol).T
  hp1, dinv = _tc_first(deg_t, x, W1)
  agg1 = _sc_agg_rows(hp1, src_t, dst_t, zrows)
  hp2 = _tc_mid(agg1, hp1, dinv, b1_2d, W2, D)
  agg2 = _sc_agg_rows(hp2, src_t, dst_t, zrows)
  h3p = _tc_mid(agg2, hp2, dinv, b2_2d, W3, 1).reshape(N)
  parts3 = _sc_agg_scalar(h3p, src_flat, dst_flat, zcol)
  return _tc_final(parts3, h3p, dinv, b3)
